# baseline (speedup 1.0000x reference)
import functools

import jax
import jax.numpy as jnp
from jax import lax
from jax.experimental import pallas as pl
from jax.experimental.pallas import tpu as pltpu
from jax.experimental.pallas import tpu_sc as plsc

N = 10000
D = 128
H = 8
DH = D // H
K = 16
L = 2
DFF = 2 * D

NPAD = 10240
RBLK = 256
SR = 16
CW = 256
ABLK = 200
EBLK = 1000
GW = 128
BPAD = NPAD * K
BIG_I32 = 2**30
F32_INF = float("inf")


def _knn_body(cq_ref, ct_ref, o_ref, d_ref):
    i = pl.program_id(0)
    nt = NPAD // CW
    iota_cw = lax.broadcasted_iota(jnp.int32, (SR, CW), 1)
    iota_sr = lax.broadcasted_iota(jnp.int32, (SR, 1), 0)

    def subgroup(sg, carry_sg):
        roff = pl.multiple_of(sg * SR, SR)
        cq = cq_ref[pl.ds(roff, SR), :]
        rowg = i * RBLK + roff + iota_sr

        for j in range(nt):
            base = j * CW
            d0 = cq[:, 0:1] - ct_ref[0:1, base:base + CW]
            d = d0 * d0
            d1 = cq[:, 1:2] - ct_ref[1:2, base:base + CW]
            d = d + d1 * d1
            d2 = cq[:, 2:3] - ct_ref[2:3, base:base + CW]
            d = d + d2 * d2
            d = jnp.where(iota_cw + base == rowg, F32_INF, d)
            d_ref[:, base:base + CW] = d

        g_prev = None
        for s in range(K // 4):
            mv = [jnp.full((SR, CW), F32_INF) for _ in range(4)]
            mc = [jnp.full((SR, CW), BIG_I32, jnp.int32) for _ in range(4)]
            for j in range(nt):
                base = j * CW
                x = d_ref[:, base:base + CW]
                cx = iota_cw + base
                if s > 0:
                    x = jnp.where(x > g_prev, x, F32_INF)
                for lvl in range(4):
                    lt = x < mv[lvl]
                    if lvl < 3:
                        x_new = jnp.where(lt, mv[lvl], x)
                        cx_new = jnp.where(lt, mc[lvl], cx)
                    mv[lvl] = jnp.where(lt, x, mv[lvl])
                    mc[lvl] = jnp.where(lt, cx, mc[lvl])
                    if lvl < 3:
                        x, cx = x_new, cx_new
            g = g_prev
            for t in range(4):
                if t == 0 and s == 0:
                    cand_v, cand_c = mv[0], mc[0]
                else:
                    if t == 0:
                        cand_v, cand_c = mv[0], mc[0]
                    else:
                        cand_v = jnp.full((SR, CW), F32_INF)
                        cand_c = jnp.full((SR, CW), BIG_I32, jnp.int32)
                        for lvl in (3, 2, 1, 0):
                            gt = mv[lvl] > g
                            cand_v = jnp.where(gt, mv[lvl], cand_v)
                            cand_c = jnp.where(gt, mc[lvl], cand_c)
                gt_new = jnp.min(cand_v, axis=1, keepdims=True)
                colt = jnp.min(jnp.where(cand_v == gt_new, cand_c, BIG_I32),
                               axis=1, keepdims=True)
                o_ref[pl.ds(roff, SR), 4 * s + t:4 * s + t + 1] = (
                    jnp.minimum(colt, N - 1))
                g = gt_new
            g_prev = g
        return carry_sg

    lax.fori_loop(0, RBLK // SR, subgroup, 0)


def _knn(c):
    ct = jnp.full((8, NPAD), 1e30, jnp.float32).at[:3, :N].set(c.T)
    cq = jnp.zeros((NPAD, 8), jnp.float32).at[:N, :3].set(c)
    return pl.pallas_call(
        _knn_body,
        grid=(NPAD // RBLK,),
        in_specs=[
            pl.BlockSpec((RBLK, 8), lambda i: (i, 0)),
            pl.BlockSpec((8, NPAD), lambda i: (0, 0)),
        ],
        out_specs=pl.BlockSpec((RBLK, K), lambda i: (i, 0)),
        out_shape=jax.ShapeDtypeStruct((NPAD, K), jnp.int32),
        scratch_shapes=[pltpu.VMEM((SR, NPAD), jnp.float32)],
    )(cq, ct)


def _embed_body(h_ref, w_ref, b_ref, o_ref):
    o_ref[...] = jnp.dot(h_ref[...], w_ref[...],
                         preferred_element_type=jnp.float32) + b_ref[...]


def _embed(h, w, b):
    return pl.pallas_call(
        _embed_body,
        grid=(N // EBLK,),
        in_specs=[
            pl.BlockSpec((EBLK, D), lambda i: (i, 0)),
            pl.BlockSpec((D, D), lambda i: (0, 0)),
            pl.BlockSpec((1, D), lambda i: (0, 0)),
        ],
        out_specs=pl.BlockSpec((EBLK, D), lambda i: (i, 0)),
        out_shape=jax.ShapeDtypeStruct((N, D), jnp.float32),
    )(h, w, b.reshape(1, D))


def _qkv_body(h_ref, wq_ref, wk_ref, wv_ref, q_ref, kv_ref):
    hb = h_ref[...]
    q_ref[...] = jnp.dot(hb, wq_ref[...], preferred_element_type=jnp.float32)
    kv_ref[:, :D] = jnp.dot(hb, wk_ref[...], preferred_element_type=jnp.float32)
    kv_ref[:, D:] = jnp.dot(hb, wv_ref[...], preferred_element_type=jnp.float32)


def _qkv(h, wq, wk, wv):
    return pl.pallas_call(
        _qkv_body,
        grid=(N // EBLK,),
        in_specs=[
            pl.BlockSpec((EBLK, D), lambda i: (i, 0)),
            pl.BlockSpec((D, D), lambda i: (0, 0)),
            pl.BlockSpec((D, D), lambda i: (0, 0)),
            pl.BlockSpec((D, D), lambda i: (0, 0)),
        ],
        out_specs=[
            pl.BlockSpec((EBLK, D), lambda i: (i, 0)),
            pl.BlockSpec((EBLK, 2 * D), lambda i: (i, 0)),
        ],
        out_shape=[
            jax.ShapeDtypeStruct((N, D), jnp.float32),
            jax.ShapeDtypeStruct((N, 2 * D), jnp.float32),
        ],
    )(h, wq, wk, wv)


def _gather(kv, idx):
    mesh = plsc.VectorSubcoreMesh(core_axis_name="core",
                                  subcore_axis_name="subcore")

    @functools.partial(
        pl.kernel,
        out_type=jax.ShapeDtypeStruct((BPAD, 2 * D), jnp.float32),
        mesh=mesh,
    )
    def gather_kernel(kv_hbm, i_hbm, o_hbm):
        def body(i_vmem, o_vmem):
            pltpu.sync_copy(kv_hbm.at[i_vmem.at[0]], o_vmem)

        pltpu.emit_pipeline(
            body,
            grid=(BPAD // GW,),
            in_specs=[pl.BlockSpec((1, GW), lambda i: (0, i))],
            out_specs=[pl.BlockSpec((GW, 2 * D), lambda i: (i, 0))],
            core_axis_name=("core", "subcore"),
            dimension_semantics=(pltpu.PARALLEL,),
        )(i_hbm, o_hbm)

    return gather_kernel(kv, idx)


def _attn_body(q_ref, kvn_ref, hin_ref, seg_ref, segt_ref, wo_ref,
               ln1s_ref, ln1b_ref, wff1_ref, bff1_ref, wff2_ref, bff2_ref,
               ln2s_ref, ln2b_ref, o_ref):
    q = q_ref[...]
    qr = jnp.broadcast_to(q.reshape(ABLK, 1, D),
                          (ABLK, K, D)).reshape(ABLK * K, D)
    kn = kvn_ref[:, :D]
    vn = kvn_ref[:, D:]
    p = qr * kn
    s8 = jnp.dot(p, seg_ref[...],
                 preferred_element_type=jnp.float32) * 0.25
    s8 = jnp.clip(s8, -5.0, 5.0)
    s3 = s8.reshape(ABLK, K, H)
    mx = jnp.max(s3, axis=1, keepdims=True)
    e = jnp.exp(s3 - mx)
    ssum = jnp.sum(e, axis=1, keepdims=True)
    a = (e / ssum).reshape(ABLK * K, H)
    ax = jnp.dot(a, segt_ref[...], preferred_element_type=jnp.float32)
    out = jnp.sum((ax * vn).reshape(ABLK, K, D), axis=1)

    h1 = hin_ref[...] + jnp.dot(out, wo_ref[...],
                                preferred_element_type=jnp.float32)
    mu = jnp.mean(h1, axis=-1, keepdims=True)
    var = jnp.mean((h1 - mu) ** 2, axis=-1, keepdims=True)
    hn = (h1 - mu) / jnp.sqrt(var + 1e-5) * ln1s_ref[...] + ln1b_ref[...]

    f = jnp.maximum(
        jnp.dot(hn, wff1_ref[...], preferred_element_type=jnp.float32)
        + bff1_ref[...], 0.0)
    f2 = jnp.dot(f, wff2_ref[...],
                 preferred_element_type=jnp.float32) + bff2_ref[...]
    h2 = hn + f2
    mu2 = jnp.mean(h2, axis=-1, keepdims=True)
    var2 = jnp.mean((h2 - mu2) ** 2, axis=-1, keepdims=True)
    o_ref[...] = ((h2 - mu2) / jnp.sqrt(var2 + 1e-5) * ln2s_ref[...]
                  + ln2b_ref[...])


def _attn(q, kvn, hin, seg, segt, wo, ln1s, ln1b, wff1, bff1, wff2, bff2,
          ln2s, ln2b):
    row = lambda i: (i, 0)
    cst = lambda i: (0, 0)
    return pl.pallas_call(
        _attn_body,
        grid=(N // ABLK,),
        in_specs=[
            pl.BlockSpec((ABLK, D), row),
            pl.BlockSpec((ABLK * K, 2 * D), row),
            pl.BlockSpec((ABLK, D), row),
            pl.BlockSpec((D, H), cst),
            pl.BlockSpec((H, D), cst),
            pl.BlockSpec((D, D), cst),
            pl.BlockSpec((1, D), cst),
            pl.BlockSpec((1, D), cst),
            pl.BlockSpec((D, DFF), cst),
            pl.BlockSpec((1, DFF), cst),
            pl.BlockSpec((DFF, D), cst),
            pl.BlockSpec((1, D), cst),
            pl.BlockSpec((1, D), cst),
            pl.BlockSpec((1, D), cst),
        ],
        out_specs=pl.BlockSpec((ABLK, D), row),
        out_shape=jax.ShapeDtypeStruct((N, D), jnp.float32),
    )(q, kvn, hin, seg, segt, wo, ln1s.reshape(1, D), ln1b.reshape(1, D),
      wff1, bff1.reshape(1, DFF), wff2, bff2.reshape(1, D),
      ln2s.reshape(1, D), ln2b.reshape(1, D))


def kernel(h, c, object_ids, W_emb, b_emb, WQ, WK, WV, WO, ln1_s, ln1_b,
           Wff1, bff1, Wff2, bff2, ln2_s, ln2_b):
    del object_ids
    neigh = _knn(c)
    idx = neigh.reshape(1, BPAD)
    seg = jnp.kron(jnp.eye(H, dtype=jnp.float32),
                   jnp.ones((DH, 1), jnp.float32))
    segt = seg.T

    hcur = _embed(h, W_emb, b_emb)
    for l in range(L):
        q, kv = _qkv(hcur, WQ[l], WK[l], WV[l])
        kvn = _gather(kv, idx)
        hcur = _attn(q, kvn, hcur, seg, segt, WO[l], ln1_s[l], ln1_b[l],
                     Wff1[l], bff1[l], Wff2[l], bff2[l], ln2_s[l], ln2_b[l])
    return hcur


# speedup vs baseline: 1.0617x; 1.0617x over previous
import functools

import jax
import jax.numpy as jnp
from jax import lax
from jax.experimental import pallas as pl
from jax.experimental.pallas import tpu as pltpu
from jax.experimental.pallas import tpu_sc as plsc

N = 10000
D = 128
H = 8
DH = D // H
K = 16
L = 2
DFF = 2 * D

NPAD = 10240
RBLK = 256
SR = 32
CW = 256
ABLK = 200
EBLK = 1000
GW = 128
BPAD = NPAD * K
BIG_I32 = 2**30
F32_INF = float("inf")


def _knn_body(cq_ref, ct_ref, o_ref, d_ref):
    i = pl.program_id(0)
    nt = NPAD // CW
    iota_cw = lax.broadcasted_iota(jnp.int32, (SR, CW), 1)
    iota_sr = lax.broadcasted_iota(jnp.int32, (SR, 1), 0)

    def subgroup(sg, carry_sg):
        roff = pl.multiple_of(sg * SR, SR)
        cq = cq_ref[pl.ds(roff, SR), :]
        rowg = i * RBLK + roff + iota_sr

        for j in range(nt):
            base = j * CW
            d0 = cq[:, 0:1] - ct_ref[0:1, base:base + CW]
            d = d0 * d0
            d1 = cq[:, 1:2] - ct_ref[1:2, base:base + CW]
            d = d + d1 * d1
            d2 = cq[:, 2:3] - ct_ref[2:3, base:base + CW]
            d = d + d2 * d2
            d = jnp.where(iota_cw + base == rowg, F32_INF, d)
            d_ref[:, base:base + CW] = d

        g_prev = None
        for s in range(K // 4):
            mv = [jnp.full((SR, CW), F32_INF) for _ in range(4)]
            mc = [jnp.full((SR, CW), BIG_I32, jnp.int32) for _ in range(4)]
            for j in range(nt):
                base = j * CW
                x = d_ref[:, base:base + CW]
                cx = iota_cw + base
                if s > 0:
                    x = jnp.where(x > g_prev, x, F32_INF)
                for lvl in range(4):
                    lt = x < mv[lvl]
                    if lvl < 3:
                        x_new = jnp.where(lt, mv[lvl], x)
                        cx_new = jnp.where(lt, mc[lvl], cx)
                    mv[lvl] = jnp.where(lt, x, mv[lvl])
                    mc[lvl] = jnp.where(lt, cx, mc[lvl])
                    if lvl < 3:
                        x, cx = x_new, cx_new
            g = g_prev
            for t in range(4):
                if t == 0 and s == 0:
                    cand_v, cand_c = mv[0], mc[0]
                else:
                    if t == 0:
                        cand_v, cand_c = mv[0], mc[0]
                    else:
                        cand_v = jnp.full((SR, CW), F32_INF)
                        cand_c = jnp.full((SR, CW), BIG_I32, jnp.int32)
                        for lvl in (3, 2, 1, 0):
                            gt = mv[lvl] > g
                            cand_v = jnp.where(gt, mv[lvl], cand_v)
                            cand_c = jnp.where(gt, mc[lvl], cand_c)
                gt_new = jnp.min(cand_v, axis=1, keepdims=True)
                colt = jnp.min(jnp.where(cand_v == gt_new, cand_c, BIG_I32),
                               axis=1, keepdims=True)
                o_ref[pl.ds(roff, SR), 4 * s + t:4 * s + t + 1] = (
                    jnp.minimum(colt, N - 1))
                g = gt_new
            g_prev = g
        return carry_sg

    lax.fori_loop(0, RBLK // SR, subgroup, 0)


def _knn(c):
    ct = jnp.full((8, NPAD), 1e30, jnp.float32).at[:3, :N].set(c.T)
    cq = jnp.zeros((NPAD, 8), jnp.float32).at[:N, :3].set(c)
    return pl.pallas_call(
        _knn_body,
        grid=(NPAD // RBLK,),
        in_specs=[
            pl.BlockSpec((RBLK, 8), lambda i: (i, 0)),
            pl.BlockSpec((8, NPAD), lambda i: (0, 0)),
        ],
        out_specs=pl.BlockSpec((RBLK, K), lambda i: (i, 0)),
        out_shape=jax.ShapeDtypeStruct((NPAD, K), jnp.int32),
        scratch_shapes=[pltpu.VMEM((SR, NPAD), jnp.float32)],
    )(cq, ct)


def _embed_body(h_ref, w_ref, b_ref, o_ref):
    o_ref[...] = jnp.dot(h_ref[...], w_ref[...],
                         preferred_element_type=jnp.float32) + b_ref[...]


def _embed(h, w, b):
    return pl.pallas_call(
        _embed_body,
        grid=(N // EBLK,),
        in_specs=[
            pl.BlockSpec((EBLK, D), lambda i: (i, 0)),
            pl.BlockSpec((D, D), lambda i: (0, 0)),
            pl.BlockSpec((1, D), lambda i: (0, 0)),
        ],
        out_specs=pl.BlockSpec((EBLK, D), lambda i: (i, 0)),
        out_shape=jax.ShapeDtypeStruct((N, D), jnp.float32),
    )(h, w, b.reshape(1, D))


def _qkv_body(h_ref, wq_ref, wk_ref, wv_ref, q_ref, kv_ref):
    hb = h_ref[...]
    q_ref[...] = jnp.dot(hb, wq_ref[...], preferred_element_type=jnp.float32)
    kv_ref[:, :D] = jnp.dot(hb, wk_ref[...], preferred_element_type=jnp.float32)
    kv_ref[:, D:] = jnp.dot(hb, wv_ref[...], preferred_element_type=jnp.float32)


def _qkv(h, wq, wk, wv):
    return pl.pallas_call(
        _qkv_body,
        grid=(N // EBLK,),
        in_specs=[
            pl.BlockSpec((EBLK, D), lambda i: (i, 0)),
            pl.BlockSpec((D, D), lambda i: (0, 0)),
            pl.BlockSpec((D, D), lambda i: (0, 0)),
            pl.BlockSpec((D, D), lambda i: (0, 0)),
        ],
        out_specs=[
            pl.BlockSpec((EBLK, D), lambda i: (i, 0)),
            pl.BlockSpec((EBLK, 2 * D), lambda i: (i, 0)),
        ],
        out_shape=[
            jax.ShapeDtypeStruct((N, D), jnp.float32),
            jax.ShapeDtypeStruct((N, 2 * D), jnp.float32),
        ],
    )(h, wq, wk, wv)


def _gather(kv, idx):
    mesh = plsc.VectorSubcoreMesh(core_axis_name="core",
                                  subcore_axis_name="subcore")

    @functools.partial(
        pl.kernel,
        out_type=jax.ShapeDtypeStruct((BPAD, 2 * D), jnp.float32),
        mesh=mesh,
    )
    def gather_kernel(kv_hbm, i_hbm, o_hbm):
        def body(i_vmem, o_vmem):
            pltpu.sync_copy(kv_hbm.at[i_vmem.at[0]], o_vmem)

        pltpu.emit_pipeline(
            body,
            grid=(BPAD // GW,),
            in_specs=[pl.BlockSpec((1, GW), lambda i: (0, i))],
            out_specs=[pl.BlockSpec((GW, 2 * D), lambda i: (i, 0))],
            core_axis_name=("core", "subcore"),
            dimension_semantics=(pltpu.PARALLEL,),
        )(i_hbm, o_hbm)

    return gather_kernel(kv, idx)


def _attn_body(q_ref, kvn_ref, hin_ref, seg_ref, segt_ref, wo_ref,
               ln1s_ref, ln1b_ref, wff1_ref, bff1_ref, wff2_ref, bff2_ref,
               ln2s_ref, ln2b_ref, o_ref):
    q = q_ref[...]
    qr = jnp.broadcast_to(q.reshape(ABLK, 1, D),
                          (ABLK, K, D)).reshape(ABLK * K, D)
    kn = kvn_ref[:, :D]
    vn = kvn_ref[:, D:]
    p = qr * kn
    s8 = jnp.dot(p, seg_ref[...],
                 preferred_element_type=jnp.float32) * 0.25
    s8 = jnp.clip(s8, -5.0, 5.0)
    s3 = s8.reshape(ABLK, K, H)
    mx = jnp.max(s3, axis=1, keepdims=True)
    e = jnp.exp(s3 - mx)
    ssum = jnp.sum(e, axis=1, keepdims=True)
    a = (e / ssum).reshape(ABLK * K, H)
    ax = jnp.dot(a, segt_ref[...], preferred_element_type=jnp.float32)
    out = jnp.sum((ax * vn).reshape(ABLK, K, D), axis=1)

    h1 = hin_ref[...] + jnp.dot(out, wo_ref[...],
                                preferred_element_type=jnp.float32)
    mu = jnp.mean(h1, axis=-1, keepdims=True)
    var = jnp.mean((h1 - mu) ** 2, axis=-1, keepdims=True)
    hn = (h1 - mu) / jnp.sqrt(var + 1e-5) * ln1s_ref[...] + ln1b_ref[...]

    f = jnp.maximum(
        jnp.dot(hn, wff1_ref[...], preferred_element_type=jnp.float32)
        + bff1_ref[...], 0.0)
    f2 = jnp.dot(f, wff2_ref[...],
                 preferred_element_type=jnp.float32) + bff2_ref[...]
    h2 = hn + f2
    mu2 = jnp.mean(h2, axis=-1, keepdims=True)
    var2 = jnp.mean((h2 - mu2) ** 2, axis=-1, keepdims=True)
    o_ref[...] = ((h2 - mu2) / jnp.sqrt(var2 + 1e-5) * ln2s_ref[...]
                  + ln2b_ref[...])


def _attn(q, kvn, hin, seg, segt, wo, ln1s, ln1b, wff1, bff1, wff2, bff2,
          ln2s, ln2b):
    row = lambda i: (i, 0)
    cst = lambda i: (0, 0)
    return pl.pallas_call(
        _attn_body,
        grid=(N // ABLK,),
        in_specs=[
            pl.BlockSpec((ABLK, D), row),
            pl.BlockSpec((ABLK * K, 2 * D), row),
            pl.BlockSpec((ABLK, D), row),
            pl.BlockSpec((D, H), cst),
            pl.BlockSpec((H, D), cst),
            pl.BlockSpec((D, D), cst),
            pl.BlockSpec((1, D), cst),
            pl.BlockSpec((1, D), cst),
            pl.BlockSpec((D, DFF), cst),
            pl.BlockSpec((1, DFF), cst),
            pl.BlockSpec((DFF, D), cst),
            pl.BlockSpec((1, D), cst),
            pl.BlockSpec((1, D), cst),
            pl.BlockSpec((1, D), cst),
        ],
        out_specs=pl.BlockSpec((ABLK, D), row),
        out_shape=jax.ShapeDtypeStruct((N, D), jnp.float32),
    )(q, kvn, hin, seg, segt, wo, ln1s.reshape(1, D), ln1b.reshape(1, D),
      wff1, bff1.reshape(1, DFF), wff2, bff2.reshape(1, D),
      ln2s.reshape(1, D), ln2b.reshape(1, D))


def kernel(h, c, object_ids, W_emb, b_emb, WQ, WK, WV, WO, ln1_s, ln1_b,
           Wff1, bff1, Wff2, bff2, ln2_s, ln2_b):
    del object_ids
    neigh = _knn(c)
    idx = neigh.reshape(1, BPAD)
    seg = jnp.kron(jnp.eye(H, dtype=jnp.float32),
                   jnp.ones((DH, 1), jnp.float32))
    segt = seg.T

    hcur = _embed(h, W_emb, b_emb)
    for l in range(L):
        q, kv = _qkv(hcur, WQ[l], WK[l], WV[l])
        kvn = _gather(kv, idx)
        hcur = _attn(q, kvn, hcur, seg, segt, WO[l], ln1_s[l], ln1_b[l],
                     Wff1[l], bff1[l], Wff2[l], bff2[l], ln2_s[l], ln2_b[l])
    return hcur


# speedup vs baseline: 1.3922x; 1.3113x over previous
import functools

import jax
import jax.numpy as jnp
from jax import lax
from jax.experimental import pallas as pl
from jax.experimental.pallas import tpu as pltpu
from jax.experimental.pallas import tpu_sc as plsc

N = 10000
D = 128
H = 8
DH = D // H
K = 16
L = 2
DFF = 2 * D

NPAD = 10240
RBLK = 256
SR = 64
CW = 256
ABLK = 200
EBLK = 1000
GW = 128
BPAD = NPAD * K
BIG_I32 = 2**30
F32_INF = float("inf")


def _knn_body(cq_ref, ct_ref, o_ref, d_ref):
    i = pl.program_id(0)
    nt = NPAD // CW
    iota_cw = lax.broadcasted_iota(jnp.int32, (SR, CW), 1)
    iota_sr = lax.broadcasted_iota(jnp.int32, (SR, 1), 0)

    def subgroup(sg, carry_sg):
        roff = pl.multiple_of(sg * SR, SR)
        cq = cq_ref[pl.ds(roff, SR), :]
        rowg = i * RBLK + roff + iota_sr

        for j in range(nt):
            base = j * CW
            d0 = cq[:, 0:1] - ct_ref[0:1, base:base + CW]
            d = d0 * d0
            d1 = cq[:, 1:2] - ct_ref[1:2, base:base + CW]
            d = d + d1 * d1
            d2 = cq[:, 2:3] - ct_ref[2:3, base:base + CW]
            d = d + d2 * d2
            d = jnp.where(iota_cw + base == rowg, F32_INF, d)
            d_ref[:, base:base + CW] = d

        m_prev = None
        for k in range(K):
            dacc = jnp.full((SR, CW), F32_INF)
            cacc = jnp.full((SR, CW), BIG_I32, jnp.int32)
            for j in range(nt):
                base = j * CW
                dt = d_ref[:, base:base + CW]
                colg = iota_cw + base
                if k == 0:
                    take = dt < dacc
                else:
                    take = (dt > m_prev) & (dt < dacc)
                dacc = jnp.where(take, dt, dacc)
                cacc = jnp.where(take, colg, cacc)
            m = jnp.min(dacc, axis=1, keepdims=True)
            idxk = jnp.min(jnp.where(dacc == m, cacc, BIG_I32), axis=1,
                           keepdims=True)
            o_ref[pl.ds(roff, SR), k:k + 1] = jnp.minimum(idxk, N - 1)
            m_prev = m
        return carry_sg

    lax.fori_loop(0, RBLK // SR, subgroup, 0)


def _knn(c):
    ct = jnp.full((8, NPAD), 1e30, jnp.float32).at[:3, :N].set(c.T)
    cq = jnp.zeros((NPAD, 8), jnp.float32).at[:N, :3].set(c)
    return pl.pallas_call(
        _knn_body,
        grid=(NPAD // RBLK,),
        in_specs=[
            pl.BlockSpec((RBLK, 8), lambda i: (i, 0)),
            pl.BlockSpec((8, NPAD), lambda i: (0, 0)),
        ],
        out_specs=pl.BlockSpec((RBLK, K), lambda i: (i, 0)),
        out_shape=jax.ShapeDtypeStruct((NPAD, K), jnp.int32),
        scratch_shapes=[pltpu.VMEM((SR, NPAD), jnp.float32)],
    )(cq, ct)


def _embed_body(h_ref, w_ref, b_ref, o_ref):
    o_ref[...] = jnp.dot(h_ref[...], w_ref[...],
                         preferred_element_type=jnp.float32) + b_ref[...]


def _embed(h, w, b):
    return pl.pallas_call(
        _embed_body,
        grid=(N // EBLK,),
        in_specs=[
            pl.BlockSpec((EBLK, D), lambda i: (i, 0)),
            pl.BlockSpec((D, D), lambda i: (0, 0)),
            pl.BlockSpec((1, D), lambda i: (0, 0)),
        ],
        out_specs=pl.BlockSpec((EBLK, D), lambda i: (i, 0)),
        out_shape=jax.ShapeDtypeStruct((N, D), jnp.float32),
    )(h, w, b.reshape(1, D))


def _qkv_body(h_ref, wq_ref, wk_ref, wv_ref, q_ref, kv_ref):
    hb = h_ref[...]
    q_ref[...] = jnp.dot(hb, wq_ref[...], preferred_element_type=jnp.float32)
    kp = jnp.dot(hb, wk_ref[...], preferred_element_type=jnp.float32)
    vp = jnp.dot(hb, wv_ref[...], preferred_element_type=jnp.float32)
    kb = lax.bitcast_convert_type(
        kp.astype(jnp.bfloat16).astype(jnp.float32), jnp.int32)
    vb = lax.bitcast_convert_type(
        vp.astype(jnp.bfloat16).astype(jnp.float32), jnp.int32)
    kv_ref[...] = lax.shift_right_logical(kb, 16) | (vb & (-65536))


def _qkv(h, wq, wk, wv):
    return pl.pallas_call(
        _qkv_body,
        grid=(N // EBLK,),
        in_specs=[
            pl.BlockSpec((EBLK, D), lambda i: (i, 0)),
            pl.BlockSpec((D, D), lambda i: (0, 0)),
            pl.BlockSpec((D, D), lambda i: (0, 0)),
            pl.BlockSpec((D, D), lambda i: (0, 0)),
        ],
        out_specs=[
            pl.BlockSpec((EBLK, D), lambda i: (i, 0)),
            pl.BlockSpec((EBLK, D), lambda i: (i, 0)),
        ],
        out_shape=[
            jax.ShapeDtypeStruct((N, D), jnp.float32),
            jax.ShapeDtypeStruct((N, D), jnp.int32),
        ],
    )(h, wq, wk, wv)


def _gather(kv, idx):
    mesh = plsc.VectorSubcoreMesh(core_axis_name="core",
                                  subcore_axis_name="subcore")

    @functools.partial(
        pl.kernel,
        out_type=jax.ShapeDtypeStruct((BPAD, D), jnp.int32),
        mesh=mesh,
    )
    def gather_kernel(kv_hbm, i_hbm, o_hbm):
        def body(i_vmem, o_vmem):
            pltpu.sync_copy(kv_hbm.at[i_vmem.at[0]], o_vmem)

        pltpu.emit_pipeline(
            body,
            grid=(BPAD // GW,),
            in_specs=[pl.BlockSpec((1, GW), lambda i: (0, i))],
            out_specs=[pl.BlockSpec((GW, D), lambda i: (i, 0))],
            core_axis_name=("core", "subcore"),
            dimension_semantics=(pltpu.PARALLEL,),
        )(i_hbm, o_hbm)

    return gather_kernel(kv, idx)


def _attn_body(q_ref, kvn_ref, hin_ref, seg_ref, segt_ref, wo_ref,
               ln1s_ref, ln1b_ref, wff1_ref, bff1_ref, wff2_ref, bff2_ref,
               ln2s_ref, ln2b_ref, o_ref):
    q = q_ref[...]
    qr = jnp.broadcast_to(q.reshape(ABLK, 1, D),
                          (ABLK, K, D)).reshape(ABLK * K, D)
    kvw = kvn_ref[...]
    kn = lax.bitcast_convert_type(lax.shift_left(kvw, 16), jnp.float32)
    vn = lax.bitcast_convert_type(kvw & (-65536), jnp.float32)
    p = qr * kn
    s8 = jnp.dot(p, seg_ref[...],
                 preferred_element_type=jnp.float32) * 0.25
    s8 = jnp.clip(s8, -5.0, 5.0)
    s3 = s8.reshape(ABLK, K, H)
    mx = jnp.max(s3, axis=1, keepdims=True)
    e = jnp.exp(s3 - mx)
    ssum = jnp.sum(e, axis=1, keepdims=True)
    a = (e / ssum).reshape(ABLK * K, H)
    ax = jnp.dot(a, segt_ref[...], preferred_element_type=jnp.float32)
    out = jnp.sum((ax * vn).reshape(ABLK, K, D), axis=1)

    h1 = hin_ref[...] + jnp.dot(out, wo_ref[...],
                                preferred_element_type=jnp.float32)
    mu = jnp.mean(h1, axis=-1, keepdims=True)
    var = jnp.mean((h1 - mu) ** 2, axis=-1, keepdims=True)
    hn = (h1 - mu) / jnp.sqrt(var + 1e-5) * ln1s_ref[...] + ln1b_ref[...]

    f = jnp.maximum(
        jnp.dot(hn, wff1_ref[...], preferred_element_type=jnp.float32)
        + bff1_ref[...], 0.0)
    f2 = jnp.dot(f, wff2_ref[...],
                 preferred_element_type=jnp.float32) + bff2_ref[...]
    h2 = hn + f2
    mu2 = jnp.mean(h2, axis=-1, keepdims=True)
    var2 = jnp.mean((h2 - mu2) ** 2, axis=-1, keepdims=True)
    o_ref[...] = ((h2 - mu2) / jnp.sqrt(var2 + 1e-5) * ln2s_ref[...]
                  + ln2b_ref[...])


def _attn(q, kvn, hin, seg, segt, wo, ln1s, ln1b, wff1, bff1, wff2, bff2,
          ln2s, ln2b):
    row = lambda i: (i, 0)
    cst = lambda i: (0, 0)
    return pl.pallas_call(
        _attn_body,
        grid=(N // ABLK,),
        in_specs=[
            pl.BlockSpec((ABLK, D), row),
            pl.BlockSpec((ABLK * K, D), row),
            pl.BlockSpec((ABLK, D), row),
            pl.BlockSpec((D, H), cst),
            pl.BlockSpec((H, D), cst),
            pl.BlockSpec((D, D), cst),
            pl.BlockSpec((1, D), cst),
            pl.BlockSpec((1, D), cst),
            pl.BlockSpec((D, DFF), cst),
            pl.BlockSpec((1, DFF), cst),
            pl.BlockSpec((DFF, D), cst),
            pl.BlockSpec((1, D), cst),
            pl.BlockSpec((1, D), cst),
            pl.BlockSpec((1, D), cst),
        ],
        out_specs=pl.BlockSpec((ABLK, D), row),
        out_shape=jax.ShapeDtypeStruct((N, D), jnp.float32),
    )(q, kvn, hin, seg, segt, wo, ln1s.reshape(1, D), ln1b.reshape(1, D),
      wff1, bff1.reshape(1, DFF), wff2, bff2.reshape(1, D),
      ln2s.reshape(1, D), ln2b.reshape(1, D))


def kernel(h, c, object_ids, W_emb, b_emb, WQ, WK, WV, WO, ln1_s, ln1_b,
           Wff1, bff1, Wff2, bff2, ln2_s, ln2_b):
    del object_ids
    neigh = _knn(c)
    idx = neigh.reshape(1, BPAD)
    seg = jnp.kron(jnp.eye(H, dtype=jnp.float32),
                   jnp.ones((DH, 1), jnp.float32))
    segt = seg.T

    hcur = _embed(h, W_emb, b_emb)
    for l in range(L):
        q, kv = _qkv(hcur, WQ[l], WK[l], WV[l])
        kvn = _gather(kv, idx)
        hcur = _attn(q, kvn, hcur, seg, segt, WO[l], ln1_s[l], ln1_b[l],
                     Wff1[l], bff1[l], Wff2[l], bff2[l], ln2_s[l], ln2_b[l])
    return hcur


# speedup vs baseline: 1.4457x; 1.0384x over previous
import functools

import jax
import jax.numpy as jnp
from jax import lax
from jax.experimental import pallas as pl
from jax.experimental.pallas import tpu as pltpu
from jax.experimental.pallas import tpu_sc as plsc

N = 10000
D = 128
H = 8
DH = D // H
K = 16
L = 2
DFF = 2 * D

NPAD = 10240
RBLK = 256
SR = 64
CW = 256
ABLK = 200
EBLK = 1000
GW = 128
NCHUNK = 5
BPAD = NPAD * K
BIG_I32 = 2**30
F32_INF = float("inf")


def _knn_body(cq_ref, ct_ref, o_ref, d_ref):
    i = pl.program_id(0)
    nt = NPAD // CW
    iota_cw = lax.broadcasted_iota(jnp.int32, (SR, CW), 1)
    iota_sr = lax.broadcasted_iota(jnp.int32, (SR, 1), 0)

    def subgroup(sg, carry_sg):
        roff = pl.multiple_of(sg * SR, SR)
        cq = cq_ref[pl.ds(roff, SR), :]
        rowg = i * RBLK + roff + iota_sr

        for j in range(nt):
            base = j * CW
            d0 = cq[:, 0:1] - ct_ref[0:1, base:base + CW]
            d = d0 * d0
            d1 = cq[:, 1:2] - ct_ref[1:2, base:base + CW]
            d = d + d1 * d1
            d2 = cq[:, 2:3] - ct_ref[2:3, base:base + CW]
            d = d + d2 * d2
            d = jnp.where(iota_cw + base == rowg, F32_INF, d)
            d_ref[:, base:base + CW] = d

        m_prev = None
        for k in range(K):
            dacc = jnp.full((SR, CW), F32_INF)
            cacc = jnp.full((SR, CW), BIG_I32, jnp.int32)
            for j in range(nt):
                base = j * CW
                dt = d_ref[:, base:base + CW]
                colg = iota_cw + base
                if k == 0:
                    take = dt < dacc
                else:
                    take = (dt > m_prev) & (dt < dacc)
                dacc = jnp.where(take, dt, dacc)
                cacc = jnp.where(take, colg, cacc)
            m = jnp.min(dacc, axis=1, keepdims=True)
            idxk = jnp.min(jnp.where(dacc == m, cacc, BIG_I32), axis=1,
                           keepdims=True)
            o_ref[pl.ds(roff, SR), k:k + 1] = jnp.minimum(idxk, N - 1)
            m_prev = m
        return carry_sg

    lax.fori_loop(0, RBLK // SR, subgroup, 0)


def _knn(c):
    ct = jnp.full((8, NPAD), 1e30, jnp.float32).at[:3, :N].set(c.T)
    cq = jnp.zeros((NPAD, 8), jnp.float32).at[:N, :3].set(c)
    return pl.pallas_call(
        _knn_body,
        grid=(NPAD // RBLK,),
        in_specs=[
            pl.BlockSpec((RBLK, 8), lambda i: (i, 0)),
            pl.BlockSpec((8, NPAD), lambda i: (0, 0)),
        ],
        out_specs=pl.BlockSpec((RBLK, K), lambda i: (i, 0)),
        out_shape=jax.ShapeDtypeStruct((NPAD, K), jnp.int32),
        scratch_shapes=[pltpu.VMEM((SR, NPAD), jnp.float32)],
    )(cq, ct)


def _embed_body(h_ref, w_ref, b_ref, o_ref):
    o_ref[...] = jnp.dot(h_ref[...], w_ref[...],
                         preferred_element_type=jnp.float32) + b_ref[...]


def _embed(h, w, b):
    return pl.pallas_call(
        _embed_body,
        grid=(N // EBLK,),
        in_specs=[
            pl.BlockSpec((EBLK, D), lambda i: (i, 0)),
            pl.BlockSpec((D, D), lambda i: (0, 0)),
            pl.BlockSpec((1, D), lambda i: (0, 0)),
        ],
        out_specs=pl.BlockSpec((EBLK, D), lambda i: (i, 0)),
        out_shape=jax.ShapeDtypeStruct((N, D), jnp.float32),
    )(h, w, b.reshape(1, D))


def _qkv_body(h_ref, wq_ref, wk_ref, wv_ref, q_ref, kv_ref):
    hb = h_ref[...]
    q_ref[...] = jnp.dot(hb, wq_ref[...], preferred_element_type=jnp.float32)
    kp = jnp.dot(hb, wk_ref[...], preferred_element_type=jnp.float32)
    vp = jnp.dot(hb, wv_ref[...], preferred_element_type=jnp.float32)
    kb = lax.bitcast_convert_type(
        kp.astype(jnp.bfloat16).astype(jnp.float32), jnp.int32)
    vb = lax.bitcast_convert_type(
        vp.astype(jnp.bfloat16).astype(jnp.float32), jnp.int32)
    kv_ref[...] = lax.shift_right_logical(kb, 16) | (vb & (-65536))


def _qkv(h, wq, wk, wv):
    return pl.pallas_call(
        _qkv_body,
        grid=(N // EBLK,),
        in_specs=[
            pl.BlockSpec((EBLK, D), lambda i: (i, 0)),
            pl.BlockSpec((D, D), lambda i: (0, 0)),
            pl.BlockSpec((D, D), lambda i: (0, 0)),
            pl.BlockSpec((D, D), lambda i: (0, 0)),
        ],
        out_specs=[
            pl.BlockSpec((EBLK, D), lambda i: (i, 0)),
            pl.BlockSpec((EBLK, D), lambda i: (i, 0)),
        ],
        out_shape=[
            jax.ShapeDtypeStruct((N, D), jnp.float32),
            jax.ShapeDtypeStruct((N, D), jnp.int32),
        ],
    )(h, wq, wk, wv)


def _gather(kv, idx, rows):
    mesh = plsc.VectorSubcoreMesh(core_axis_name="core",
                                  subcore_axis_name="subcore")

    @functools.partial(
        pl.kernel,
        out_type=jax.ShapeDtypeStruct((rows, D), jnp.int32),
        mesh=mesh,
    )
    def gather_kernel(kv_hbm, i_hbm, o_hbm):
        def body(i_vmem, o_vmem):
            pltpu.sync_copy(kv_hbm.at[i_vmem.at[0]], o_vmem)

        pltpu.emit_pipeline(
            body,
            grid=(rows // GW,),
            in_specs=[pl.BlockSpec((1, GW), lambda i: (0, i))],
            out_specs=[pl.BlockSpec((GW, D), lambda i: (i, 0))],
            core_axis_name=("core", "subcore"),
            dimension_semantics=(pltpu.PARALLEL,),
        )(i_hbm, o_hbm)

    return gather_kernel(kv, idx)


def _attn_body(q_ref, kvn_ref, hin_ref, seg_ref, segt_ref, wo_ref,
               ln1s_ref, ln1b_ref, wff1_ref, bff1_ref, wff2_ref, bff2_ref,
               ln2s_ref, ln2b_ref, o_ref):
    q = q_ref[...]
    qr = jnp.broadcast_to(q.reshape(ABLK, 1, D),
                          (ABLK, K, D)).reshape(ABLK * K, D)
    kvw = kvn_ref[...]
    kn = lax.bitcast_convert_type(lax.shift_left(kvw, 16), jnp.float32)
    vn = lax.bitcast_convert_type(kvw & (-65536), jnp.float32)
    p = qr * kn
    s8 = jnp.dot(p, seg_ref[...],
                 preferred_element_type=jnp.float32) * 0.25
    s8 = jnp.clip(s8, -5.0, 5.0)
    s3 = s8.reshape(ABLK, K, H)
    mx = jnp.max(s3, axis=1, keepdims=True)
    e = jnp.exp(s3 - mx)
    ssum = jnp.sum(e, axis=1, keepdims=True)
    a = (e / ssum).reshape(ABLK * K, H)
    ax = jnp.dot(a, segt_ref[...], preferred_element_type=jnp.float32)
    out = jnp.sum((ax * vn).reshape(ABLK, K, D), axis=1)

    h1 = hin_ref[...] + jnp.dot(out, wo_ref[...],
                                preferred_element_type=jnp.float32)
    mu = jnp.mean(h1, axis=-1, keepdims=True)
    var = jnp.mean((h1 - mu) ** 2, axis=-1, keepdims=True)
    hn = (h1 - mu) / jnp.sqrt(var + 1e-5) * ln1s_ref[...] + ln1b_ref[...]

    f = jnp.maximum(
        jnp.dot(hn, wff1_ref[...], preferred_element_type=jnp.float32)
        + bff1_ref[...], 0.0)
    f2 = jnp.dot(f, wff2_ref[...],
                 preferred_element_type=jnp.float32) + bff2_ref[...]
    h2 = hn + f2
    mu2 = jnp.mean(h2, axis=-1, keepdims=True)
    var2 = jnp.mean((h2 - mu2) ** 2, axis=-1, keepdims=True)
    o_ref[...] = ((h2 - mu2) / jnp.sqrt(var2 + 1e-5) * ln2s_ref[...]
                  + ln2b_ref[...])


def _attn(q, kvn, hin, seg, segt, wo, ln1s, ln1b, wff1, bff1, wff2, bff2,
          ln2s, ln2b, rows):
    row = lambda i: (i, 0)
    cst = lambda i: (0, 0)
    return pl.pallas_call(
        _attn_body,
        grid=(rows // ABLK,),
        in_specs=[
            pl.BlockSpec((ABLK, D), row),
            pl.BlockSpec((ABLK * K, D), row),
            pl.BlockSpec((ABLK, D), row),
            pl.BlockSpec((D, H), cst),
            pl.BlockSpec((H, D), cst),
            pl.BlockSpec((D, D), cst),
            pl.BlockSpec((1, D), cst),
            pl.BlockSpec((1, D), cst),
            pl.BlockSpec((D, DFF), cst),
            pl.BlockSpec((1, DFF), cst),
            pl.BlockSpec((DFF, D), cst),
            pl.BlockSpec((1, D), cst),
            pl.BlockSpec((1, D), cst),
            pl.BlockSpec((1, D), cst),
        ],
        out_specs=pl.BlockSpec((ABLK, D), row),
        out_shape=jax.ShapeDtypeStruct((rows, D), jnp.float32),
    )(q, kvn, hin, seg, segt, wo, ln1s.reshape(1, D), ln1b.reshape(1, D),
      wff1, bff1.reshape(1, DFF), wff2, bff2.reshape(1, D),
      ln2s.reshape(1, D), ln2b.reshape(1, D))


def kernel(h, c, object_ids, W_emb, b_emb, WQ, WK, WV, WO, ln1_s, ln1_b,
           Wff1, bff1, Wff2, bff2, ln2_s, ln2_b):
    del object_ids
    neigh = _knn(c)
    idx = neigh.reshape(1, BPAD)
    seg = jnp.kron(jnp.eye(H, dtype=jnp.float32),
                   jnp.ones((DH, 1), jnp.float32))
    segt = seg.T

    hcur = _embed(h, W_emb, b_emb)
    cn = N // NCHUNK
    ec = cn * K
    for l in range(L):
        q, kv = _qkv(hcur, WQ[l], WK[l], WV[l])
        outs = []
        for c in range(NCHUNK):
            kvn = _gather(kv, idx[:, c * ec:(c + 1) * ec], ec)
            outs.append(_attn(
                q[c * cn:(c + 1) * cn], kvn, hcur[c * cn:(c + 1) * cn],
                seg, segt, WO[l], ln1_s[l], ln1_b[l], Wff1[l], bff1[l],
                Wff2[l], bff2[l], ln2_s[l], ln2_b[l], cn))
        hcur = jnp.concatenate(outs, axis=0)
    return hcur
